# baseline (speedup 1.0000x reference)
import jax
import jax.numpy as jnp
from jax.experimental import pallas as pl
from jax.experimental.pallas import tpu as pltpu

_N, _F_OUT = 10000, 64
_K, _ALPHA = 10, 0.1
_QSCALE = 32768.0
_F4 = jnp.float4_e2m1fn
_F8 = jnp.float8_e4m3fn
_BI = 1024
_GRID_I = (_N + _BI - 1) // _BI
_NPAD = _GRID_I * _BI
_BI1 = 512
_GRID_I1 = (_N + _BI1 - 1) // _BI1
_BN_MLP = 2000


def _mlp_body(x_ref, w1t_ref, b1_ref, w2t_ref, b2_ref, h_ref, hmax_ref):
    a = jax.lax.dot_general(x_ref[...], w1t_ref[...], (((1,), (0,)), ((), ())),
                            preferred_element_type=jnp.float32)
    a = jnp.maximum(a + b1_ref[...], 0.0)
    h = jax.lax.dot_general(a, w2t_ref[...], (((1,), (0,)), ((), ())),
                            preferred_element_type=jnp.float32) + b2_ref[...]
    h_ref[...] = h
    bmax = jnp.max(jnp.abs(h), axis=0, keepdims=True)
    i = pl.program_id(0)

    @pl.when(i == 0)
    def _():
        hmax_ref[...] = bmax

    @pl.when(i != 0)
    def _():
        hmax_ref[...] = jnp.maximum(hmax_ref[...], bmax)


def _pass1_body(adj_ref, h_all_ref, h_blk_ref, out_ref, q_ref, omax_ref, cs_ref):
    i = pl.program_id(0)
    a = adj_ref[...]
    q_ref[...] = ((a - 1.0 / _N) * _QSCALE).astype(_F4)
    acc = jax.lax.dot_general(a, h_all_ref[...], (((1,), (0,)), ((), ())),
                              preferred_element_type=jnp.float32)
    out = (1.0 - _ALPHA) * acc + _ALPHA * h_blk_ref[...]
    out_ref[...] = out
    row = jax.lax.broadcasted_iota(jnp.int32, (_BI1, 1), 0) + i * _BI1
    valid = row < _N
    bmax = jnp.max(jnp.where(valid, jnp.abs(out), 0.0), axis=0, keepdims=True)
    bsum = jnp.sum(jnp.where(valid, out, 0.0), axis=0, keepdims=True)

    @pl.when(i == 0)
    def _():
        omax_ref[...] = bmax
        cs_ref[...] = bsum

    @pl.when(i != 0)
    def _():
        omax_ref[...] = jnp.maximum(omax_ref[...], bmax)
        cs_ref[...] = cs_ref[...] + bsum


def _prop_body(q_ref, out1_ref, h_blk_ref, so09_ref, co_ref, cs1_ref,
               out_ref, qo_ref, cs_ref):
    k = pl.program_id(0)
    i = pl.program_id(1)

    @pl.when(jnp.logical_and(k == 0, i == 0))
    def _():
        qo_ref[0, pl.ds(0, _N), :] = (out1_ref[...] * co_ref[...]).astype(_F8)
        cs_ref[0] = cs1_ref[...]

    cur = jax.lax.rem(k, 2)
    qo = qo_ref[cur, pl.ds(0, _N), :]
    acc = jax.lax.dot_general(q_ref[...], qo, (((1,), (0,)), ((), ())),
                              preferred_element_type=jnp.float32)
    out = (acc * so09_ref[...]
           + ((1.0 - _ALPHA) / _N) * cs_ref[cur]
           + _ALPHA * h_blk_ref[...])
    out_ref[...] = out
    qo_ref[1 - cur, pl.ds(i * _BI, _BI), :] = (out * co_ref[...]).astype(_F8)
    row = jax.lax.broadcasted_iota(jnp.int32, (_BI, 1), 0) + i * _BI
    bsum = jnp.sum(jnp.where(row < _N, out, 0.0), axis=0, keepdims=True)

    @pl.when(i == 0)
    def _():
        cs_ref[1 - cur] = bsum

    @pl.when(i != 0)
    def _():
        cs_ref[1 - cur] = cs_ref[1 - cur] + bsum


def kernel(x, adj, W1, b1, W2, b2):
    n, f_in = x.shape
    hdim = W1.shape[0]
    f_out = W2.shape[0]

    h, hmax = pl.pallas_call(
        _mlp_body,
        grid=(n // _BN_MLP,),
        in_specs=[
            pl.BlockSpec((_BN_MLP, f_in), lambda i: (i, 0)),
            pl.BlockSpec((f_in, hdim), lambda i: (0, 0)),
            pl.BlockSpec((1, hdim), lambda i: (0, 0)),
            pl.BlockSpec((hdim, f_out), lambda i: (0, 0)),
            pl.BlockSpec((1, f_out), lambda i: (0, 0)),
        ],
        out_specs=[
            pl.BlockSpec((_BN_MLP, f_out), lambda i: (i, 0)),
            pl.BlockSpec((1, f_out), lambda i: (0, 0)),
        ],
        out_shape=[
            jax.ShapeDtypeStruct((n, f_out), jnp.float32),
            jax.ShapeDtypeStruct((1, f_out), jnp.float32),
        ],
    )(x, W1.T, b1.reshape(1, -1), W2.T, b2.reshape(1, -1))

    out1, q, omax, cs1 = pl.pallas_call(
        _pass1_body,
        grid=(_GRID_I1,),
        in_specs=[
            pl.BlockSpec((_BI1, n), lambda i: (i, 0)),
            pl.BlockSpec((n, f_out), lambda i: (0, 0)),
            pl.BlockSpec((_BI1, f_out), lambda i: (i, 0)),
        ],
        out_specs=[
            pl.BlockSpec((_BI1, f_out), lambda i: (i, 0)),
            pl.BlockSpec((_BI1, n), lambda i: (i, 0)),
            pl.BlockSpec((1, f_out), lambda i: (0, 0)),
            pl.BlockSpec((1, f_out), lambda i: (0, 0)),
        ],
        out_shape=[
            jax.ShapeDtypeStruct((n, f_out), jnp.float32),
            jax.ShapeDtypeStruct((n, n), _F4),
            jax.ShapeDtypeStruct((1, f_out), jnp.float32),
            jax.ShapeDtypeStruct((1, f_out), jnp.float32),
        ],
    )(adj, h, h)

    so = jnp.maximum(omax, hmax)
    co = 1.0 / so
    so09 = (1.0 - _ALPHA) * so / _QSCALE

    out = pl.pallas_call(
        _prop_body,
        grid=(_K - 1, _GRID_I),
        in_specs=[
            pl.BlockSpec((_BI, n), lambda k, i: (i, 0)),
            pl.BlockSpec((n, f_out), lambda k, i: (0, 0)),
            pl.BlockSpec((_BI, f_out), lambda k, i: (i, 0)),
            pl.BlockSpec((1, f_out), lambda k, i: (0, 0)),
            pl.BlockSpec((1, f_out), lambda k, i: (0, 0)),
            pl.BlockSpec((1, f_out), lambda k, i: (0, 0)),
        ],
        out_specs=pl.BlockSpec((_BI, f_out), lambda k, i: (i, 0)),
        out_shape=jax.ShapeDtypeStruct((n, f_out), jnp.float32),
        scratch_shapes=[
            pltpu.VMEM((2, _NPAD, f_out), _F8),
            pltpu.VMEM((2, 1, f_out), jnp.float32),
        ],
    )(q, out1, h, so09, co, cs1)
    return out


# speedup vs baseline: 1.0139x; 1.0139x over previous
import jax
import jax.numpy as jnp
from jax.experimental import pallas as pl
from jax.experimental.pallas import tpu as pltpu

_N, _F_OUT = 10000, 64
_K, _ALPHA = 10, 0.1
_QSCALE = 32768.0
_F4 = jnp.float4_e2m1fn
_F8 = jnp.float8_e4m3fn
_BI = 1024
_GRID_I = (_N + _BI - 1) // _BI
_NPAD = _GRID_I * _BI
_BI1 = 448
_GRID_I1 = (_N + _BI1 - 1) // _BI1
_BN_MLP = 2000


def _pass1_body(x_ref, w1t_ref, b1_ref, w2t_ref, b2_ref, adj_ref,
                out_ref, h_ref, hmax_ref, omax_ref, cs_ref, q_ref, hs_ref):
    i = pl.program_id(0)

    @pl.when(i == 0)
    def _():
        t = jax.lax.dot_general(x_ref[...], w1t_ref[...], (((1,), (0,)), ((), ())),
                                preferred_element_type=jnp.float32)
        t = jnp.maximum(t + b1_ref[...], 0.0)
        hv = jax.lax.dot_general(t, w2t_ref[...], (((1,), (0,)), ((), ())),
                                 preferred_element_type=jnp.float32) + b2_ref[...]
        hs_ref[pl.ds(0, _N), :] = hv
        h_ref[...] = hv
        hmax_ref[...] = jnp.max(jnp.abs(hv), axis=0, keepdims=True)

    a = adj_ref[...]
    q_ref[...] = ((a - 1.0 / _N) * _QSCALE).astype(_F4)
    acc = jax.lax.dot_general(a, hs_ref[pl.ds(0, _N), :], (((1,), (0,)), ((), ())),
                              preferred_element_type=jnp.float32)
    out = (1.0 - _ALPHA) * acc + _ALPHA * hs_ref[pl.ds(i * _BI1, _BI1), :]
    out_ref[...] = out
    row = jax.lax.broadcasted_iota(jnp.int32, (_BI1, 1), 0) + i * _BI1
    valid = row < _N
    bmax = jnp.max(jnp.where(valid, jnp.abs(out), 0.0), axis=0, keepdims=True)
    bsum = jnp.sum(jnp.where(valid, out, 0.0), axis=0, keepdims=True)

    @pl.when(i == 0)
    def _():
        omax_ref[...] = bmax
        cs_ref[...] = bsum

    @pl.when(i != 0)
    def _():
        omax_ref[...] = jnp.maximum(omax_ref[...], bmax)
        cs_ref[...] = cs_ref[...] + bsum


def _prop_body(q_ref, out1_ref, h_blk_ref, so09_ref, co_ref, cs1_ref,
               out_ref, qo_ref, cs_ref):
    k = pl.program_id(0)
    i = pl.program_id(1)

    @pl.when(jnp.logical_and(k == 0, i == 0))
    def _():
        qo_ref[0, pl.ds(0, _N), :] = (out1_ref[...] * co_ref[...]).astype(_F8)
        cs_ref[0] = cs1_ref[...]

    cur = jax.lax.rem(k, 2)
    qo = qo_ref[cur, pl.ds(0, _N), :]
    acc = jax.lax.dot_general(q_ref[...], qo, (((1,), (0,)), ((), ())),
                              preferred_element_type=jnp.float32)
    out = (acc * so09_ref[...]
           + ((1.0 - _ALPHA) / _N) * cs_ref[cur]
           + _ALPHA * h_blk_ref[...])
    out_ref[...] = out
    qo_ref[1 - cur, pl.ds(i * _BI, _BI), :] = (out * co_ref[...]).astype(_F8)
    row = jax.lax.broadcasted_iota(jnp.int32, (_BI, 1), 0) + i * _BI
    bsum = jnp.sum(jnp.where(row < _N, out, 0.0), axis=0, keepdims=True)

    @pl.when(i == 0)
    def _():
        cs_ref[1 - cur] = bsum

    @pl.when(i != 0)
    def _():
        cs_ref[1 - cur] = cs_ref[1 - cur] + bsum


def kernel(x, adj, W1, b1, W2, b2):
    n, f_in = x.shape
    hdim = W1.shape[0]
    f_out = W2.shape[0]

    out1, h, hmax, omax, cs1, q = pl.pallas_call(
        _pass1_body,
        grid=(_GRID_I1,),
        in_specs=[
            pl.BlockSpec((n, f_in), lambda i: (0, 0)),
            pl.BlockSpec((f_in, hdim), lambda i: (0, 0)),
            pl.BlockSpec((1, hdim), lambda i: (0, 0)),
            pl.BlockSpec((hdim, f_out), lambda i: (0, 0)),
            pl.BlockSpec((1, f_out), lambda i: (0, 0)),
            pl.BlockSpec((_BI1, n), lambda i: (i, 0)),
        ],
        out_specs=[
            pl.BlockSpec((_BI1, f_out), lambda i: (i, 0)),
            pl.BlockSpec((n, f_out), lambda i: (0, 0)),
            pl.BlockSpec((1, f_out), lambda i: (0, 0)),
            pl.BlockSpec((1, f_out), lambda i: (0, 0)),
            pl.BlockSpec((1, f_out), lambda i: (0, 0)),
            pl.BlockSpec((_BI1, n), lambda i: (i, 0)),
        ],
        out_shape=[
            jax.ShapeDtypeStruct((n, f_out), jnp.float32),
            jax.ShapeDtypeStruct((n, f_out), jnp.float32),
            jax.ShapeDtypeStruct((1, f_out), jnp.float32),
            jax.ShapeDtypeStruct((1, f_out), jnp.float32),
            jax.ShapeDtypeStruct((1, f_out), jnp.float32),
            jax.ShapeDtypeStruct((n, n), _F4),
        ],
        scratch_shapes=[pltpu.VMEM((_GRID_I1 * _BI1, f_out), jnp.float32)],
    )(x, W1.T, b1.reshape(1, -1), W2.T, b2.reshape(1, -1), adj)

    so = jnp.maximum(omax, hmax)
    co = 1.0 / so
    so09 = (1.0 - _ALPHA) * so / _QSCALE

    out = pl.pallas_call(
        _prop_body,
        grid=(_K - 1, _GRID_I),
        in_specs=[
            pl.BlockSpec((_BI, n), lambda k, i: (i, 0)),
            pl.BlockSpec((n, f_out), lambda k, i: (0, 0)),
            pl.BlockSpec((_BI, f_out), lambda k, i: (i, 0)),
            pl.BlockSpec((1, f_out), lambda k, i: (0, 0)),
            pl.BlockSpec((1, f_out), lambda k, i: (0, 0)),
            pl.BlockSpec((1, f_out), lambda k, i: (0, 0)),
        ],
        out_specs=pl.BlockSpec((_BI, f_out), lambda k, i: (i, 0)),
        out_shape=jax.ShapeDtypeStruct((n, f_out), jnp.float32),
        scratch_shapes=[
            pltpu.VMEM((2, _NPAD, f_out), _F8),
            pltpu.VMEM((2, 1, f_out), jnp.float32),
        ],
    )(q, out1, h, so09, co, cs1)
    return out


# speedup vs baseline: 1.0144x; 1.0006x over previous
import jax
import jax.numpy as jnp
from jax.experimental import pallas as pl
from jax.experimental.pallas import tpu as pltpu

_N, _F_OUT = 10000, 64
_K, _ALPHA = 10, 0.1
_QSCALE = 32768.0
_F4 = jnp.float4_e2m1fn
_F8 = jnp.float8_e4m3fn
_BI = 1024
_GRID_I = (_N + _BI - 1) // _BI
_NPAD = _GRID_I * _BI
_BI1 = 480
_GRID_I1 = (_N + _BI1 - 1) // _BI1
_BN_MLP = 2000


def _pass1_body(x_ref, w1t_ref, b1_ref, w2t_ref, b2_ref, adj_ref,
                out_ref, h_ref, hmax_ref, omax_ref, cs_ref, q_ref, hs_ref):
    i = pl.program_id(0)

    @pl.when(i == 0)
    def _():
        t = jax.lax.dot_general(x_ref[...], w1t_ref[...], (((1,), (0,)), ((), ())),
                                preferred_element_type=jnp.float32)
        t = jnp.maximum(t + b1_ref[...], 0.0)
        hv = jax.lax.dot_general(t, w2t_ref[...], (((1,), (0,)), ((), ())),
                                 preferred_element_type=jnp.float32) + b2_ref[...]
        hs_ref[pl.ds(0, _N), :] = hv
        h_ref[...] = hv
        hmax_ref[...] = jnp.max(jnp.abs(hv), axis=0, keepdims=True)

    a = adj_ref[...]
    q_ref[...] = ((a - 1.0 / _N) * _QSCALE).astype(_F4)
    acc = jax.lax.dot_general(a, hs_ref[pl.ds(0, _N), :], (((1,), (0,)), ((), ())),
                              preferred_element_type=jnp.float32)
    out = (1.0 - _ALPHA) * acc + _ALPHA * hs_ref[pl.ds(i * _BI1, _BI1), :]
    out_ref[...] = out
    row = jax.lax.broadcasted_iota(jnp.int32, (_BI1, 1), 0) + i * _BI1
    valid = row < _N
    bmax = jnp.max(jnp.where(valid, jnp.abs(out), 0.0), axis=0, keepdims=True)
    bsum = jnp.sum(jnp.where(valid, out, 0.0), axis=0, keepdims=True)

    @pl.when(i == 0)
    def _():
        omax_ref[...] = bmax
        cs_ref[...] = bsum

    @pl.when(i != 0)
    def _():
        omax_ref[...] = jnp.maximum(omax_ref[...], bmax)
        cs_ref[...] = cs_ref[...] + bsum


def _prop_body(q_ref, out1_ref, h_blk_ref, so09_ref, co_ref, cs1_ref,
               out_ref, qo_ref, cs_ref):
    k = pl.program_id(0)
    i = pl.program_id(1)

    @pl.when(jnp.logical_and(k == 0, i == 0))
    def _():
        qo_ref[0, pl.ds(0, _N), :] = (out1_ref[...] * co_ref[...]).astype(_F8)
        cs_ref[0] = cs1_ref[...]

    cur = jax.lax.rem(k, 2)
    qo = qo_ref[cur, pl.ds(0, _N), :]
    acc = jax.lax.dot_general(q_ref[...], qo, (((1,), (0,)), ((), ())),
                              preferred_element_type=jnp.float32)
    out = (acc * so09_ref[...]
           + ((1.0 - _ALPHA) / _N) * cs_ref[cur]
           + _ALPHA * h_blk_ref[...])
    out_ref[...] = out
    qo_ref[1 - cur, pl.ds(i * _BI, _BI), :] = (out * co_ref[...]).astype(_F8)
    row = jax.lax.broadcasted_iota(jnp.int32, (_BI, 1), 0) + i * _BI
    bsum = jnp.sum(jnp.where(row < _N, out, 0.0), axis=0, keepdims=True)

    @pl.when(i == 0)
    def _():
        cs_ref[1 - cur] = bsum

    @pl.when(i != 0)
    def _():
        cs_ref[1 - cur] = cs_ref[1 - cur] + bsum


def kernel(x, adj, W1, b1, W2, b2):
    n, f_in = x.shape
    hdim = W1.shape[0]
    f_out = W2.shape[0]

    out1, h, hmax, omax, cs1, q = pl.pallas_call(
        _pass1_body,
        grid=(_GRID_I1,),
        in_specs=[
            pl.BlockSpec((n, f_in), lambda i: (0, 0)),
            pl.BlockSpec((f_in, hdim), lambda i: (0, 0)),
            pl.BlockSpec((1, hdim), lambda i: (0, 0)),
            pl.BlockSpec((hdim, f_out), lambda i: (0, 0)),
            pl.BlockSpec((1, f_out), lambda i: (0, 0)),
            pl.BlockSpec((_BI1, n), lambda i: (i, 0)),
        ],
        out_specs=[
            pl.BlockSpec((_BI1, f_out), lambda i: (i, 0)),
            pl.BlockSpec((n, f_out), lambda i: (0, 0)),
            pl.BlockSpec((1, f_out), lambda i: (0, 0)),
            pl.BlockSpec((1, f_out), lambda i: (0, 0)),
            pl.BlockSpec((1, f_out), lambda i: (0, 0)),
            pl.BlockSpec((_BI1, n), lambda i: (i, 0)),
        ],
        out_shape=[
            jax.ShapeDtypeStruct((n, f_out), jnp.float32),
            jax.ShapeDtypeStruct((n, f_out), jnp.float32),
            jax.ShapeDtypeStruct((1, f_out), jnp.float32),
            jax.ShapeDtypeStruct((1, f_out), jnp.float32),
            jax.ShapeDtypeStruct((1, f_out), jnp.float32),
            jax.ShapeDtypeStruct((n, n), _F4),
        ],
        scratch_shapes=[pltpu.VMEM((_GRID_I1 * _BI1, f_out), jnp.float32)],
    )(x, W1.T, b1.reshape(1, -1), W2.T, b2.reshape(1, -1), adj)

    so = jnp.maximum(omax, hmax)
    co = 1.0 / so
    so09 = (1.0 - _ALPHA) * so / _QSCALE

    out = pl.pallas_call(
        _prop_body,
        grid=(_K - 1, _GRID_I),
        in_specs=[
            pl.BlockSpec((_BI, n), lambda k, i: (i, 0)),
            pl.BlockSpec((n, f_out), lambda k, i: (0, 0)),
            pl.BlockSpec((_BI, f_out), lambda k, i: (i, 0)),
            pl.BlockSpec((1, f_out), lambda k, i: (0, 0)),
            pl.BlockSpec((1, f_out), lambda k, i: (0, 0)),
            pl.BlockSpec((1, f_out), lambda k, i: (0, 0)),
        ],
        out_specs=pl.BlockSpec((_BI, f_out), lambda k, i: (i, 0)),
        out_shape=jax.ShapeDtypeStruct((n, f_out), jnp.float32),
        scratch_shapes=[
            pltpu.VMEM((2, _NPAD, f_out), _F8),
            pltpu.VMEM((2, 1, f_out), jnp.float32),
        ],
    )(q, out1, h, so09, co, cs1)
    return out
